# baseline (speedup 1.0000x reference)
import jax
import jax.numpy as jnp
from jax import lax
from jax.experimental import pallas as pl
from jax.experimental.pallas import tpu as pltpu
from jax.experimental.pallas import tpu_sc as plsc

H = 256
F32 = jnp.float32
NT = 16

N_COMP, N_PIN, N_SUB, N_NET = 10000, 50000, 2000, 20000
NP_COMP, NP_PIN, NP_SUB, NP_NET = 10240, 51200, 2048, 20480


def _ru(x, m):
    return (x + m - 1) // m * m


def _make_layer_sc(rels):
    mesh = plsc.VectorSubcoreMesh(core_axis_name="c", subcore_axis_name="s")
    max_et = max(ep // NT for _, ep in rels)
    max_nbt = max_et // 128
    ZR = 640

    def body(*refs):
        n = len(rels)
        srcs = refs[0:n]
        sidxs = refs[n:2 * n]
        didxs = refs[2 * n:3 * n]
        outs = refs[3 * n:4 * n]
        (sidx_v, didx_v, r0_v, r1_v, r2_v, r3_v, zbuf_v, acc_sh,
         gsem, ssem, zsem) = refs[4 * n:]
        rows = (r0_v, r1_v, r2_v, r3_v)
        cid = lax.axis_index("c")
        sid = lax.axis_index("s")
        zv = jnp.zeros((16,), F32)

        def zb_fill(i, carry):
            zbuf_v[i, :] = zv
            return carry
        lax.fori_loop(0, ZR, zb_fill, 0)

        for r in range(n):
            N_acc, E_pad = rels[r]
            et = E_pad // NT
            nbt = et // 128
            stripe = N_acc // NT
            zrows = min(stripe, ZR)
            nzc = stripe // zrows
            blocks = [(k * 4, 4) for k in range(nbt // 4)]
            if nbt % 4:
                blocks.append((nbt // 4 * 4, nbt % 4))
            nblk = len(blocks)
            base = cid * 8

            pltpu.sync_copy(didxs[r].at[sid], didx_v.at[pl.ds(0, nbt)])

            def gather(k):
                r0, nr = blocks[k]
                return pltpu.async_copy(
                    srcs[r].at[sidx_v.at[pl.ds(r0 * 128, nr * 128)]],
                    rows[k % 4].at[pl.ds(0, nr * 128)], gsem)

            def scatter(k):
                r0, nr = blocks[k]
                return [pltpu.async_copy(
                    rows[k % 4].at[pl.ds(rr * 128, 128)],
                    acc_sh.at[didx_v.at[r0 + rr]], ssem, add=True)
                    for rr in range(nr)]

            def chunk_body(j, carry):
                cc = base + j
                pltpu.sync_copy(sidxs[r].at[cc, sid],
                                sidx_v.at[pl.ds(0, et)])
                gd = [gather(k) for k in range(min(3, nblk))]
                gd += [None] * (nblk - len(gd))
                cp = base + lax.rem(j + 7, 8)
                pltpu.sync_copy(
                    acc_sh.at[pl.ds(sid * stripe, stripe)],
                    outs[r].at[pl.ds(sid * stripe, stripe),
                               pl.ds(cp * 16, 16)])
                zd = [pltpu.async_copy(
                    zbuf_v.at[pl.ds(0, zrows)],
                    acc_sh.at[pl.ds(sid * stripe + z * zrows, zrows)], zsem)
                    for z in range(nzc)]
                for d in zd:
                    d.wait()
                plsc.subcore_barrier()
                sd = [None] * nblk
                for k in range(nblk):
                    gd[k].wait()
                    sd[k] = scatter(k)
                    nx = k + 3
                    if nx < nblk:
                        if nx - 4 >= 0:
                            for d in sd[nx - 4]:
                                d.wait()
                        gd[nx] = gather(nx)
                for k in range(max(0, nblk - 4), nblk):
                    for d in sd[k]:
                        d.wait()
                plsc.subcore_barrier()
                return carry

            lax.fori_loop(0, 8, chunk_body, 0)
            pltpu.sync_copy(
                acc_sh.at[pl.ds(sid * stripe, stripe)],
                outs[r].at[pl.ds(sid * stripe, stripe),
                           pl.ds((base + 7) * 16, 16)])
            plsc.subcore_barrier()

    return pl.kernel(
        body,
        out_type=tuple(jax.ShapeDtypeStruct((na, H), F32) for na, _ in rels),
        mesh=mesh,
        compiler_params=pltpu.CompilerParams(use_tc_tiling_on_sc=False),
        scratch_types=[
            pltpu.VMEM((max_et,), jnp.int32),
            pltpu.VMEM((max_nbt, 128), jnp.int32),
            pltpu.VMEM((512, 16), F32),
            pltpu.VMEM((512, 16), F32),
            pltpu.VMEM((512, 16), F32),
            pltpu.VMEM((512, 16), F32),
            pltpu.VMEM((640, 16), F32),
            pltpu.VMEM_SHARED((max(na for na, _ in rels), 16), F32),
            pltpu.SemaphoreType.DMA,
            pltpu.SemaphoreType.DMA,
            pltpu.SemaphoreType.DMA,
        ],
    )


def _make_deg(N_acc, E_pad):
    nw = 32
    NBt = E_pad // nw // 128
    stripe = N_acc // NT
    zrows = min(stripe, 640)
    nzc = stripe // zrows
    mesh = plsc.VectorSubcoreMesh(core_axis_name="c", subcore_axis_name="s")

    def body(ones_hbm, didx_hbm, out_hbm, didx_v, rows_v, zbuf_v, acc_sh,
             gsem, ssem):
        cid = lax.axis_index("c")
        sid = lax.axis_index("s")
        w = sid * 2 + cid
        zv = jnp.zeros((16,), F32)

        def zb_fill(i, carry):
            zbuf_v[i, :] = zv
            return carry
        lax.fori_loop(0, zrows, zb_fill, 0)

        pltpu.sync_copy(didx_hbm.at[w], didx_v)
        for z in range(nzc):
            pltpu.sync_copy(
                zbuf_v.at[pl.ds(0, zrows)],
                acc_sh.at[pl.ds(sid * stripe + z * zrows, zrows)])
        plsc.subcore_barrier()
        pltpu.sync_copy(ones_hbm.at[pl.ds(0, 128)], rows_v)
        sd = []
        for b in range(NBt):
            sd.append(pltpu.async_copy(
                rows_v, acc_sh.at[didx_v.at[b]], ssem, add=True))
        for d in sd:
            d.wait()
        plsc.subcore_barrier()
        pltpu.sync_copy(
            acc_sh.at[pl.ds(sid * stripe, stripe)],
            out_hbm.at[cid, pl.ds(sid * stripe, stripe)])

    return pl.kernel(
        body,
        out_type=jax.ShapeDtypeStruct((2, N_acc, 16), F32),
        mesh=mesh,
        compiler_params=pltpu.CompilerParams(use_tc_tiling_on_sc=False),
        scratch_types=[
            pltpu.VMEM((NBt, 128), jnp.int32),
            pltpu.VMEM((128, 16), F32),
            pltpu.VMEM((640, 16), F32),
            pltpu.VMEM_SHARED((N_acc, 16), F32),
            pltpu.SemaphoreType.DMA,
            pltpu.SemaphoreType.DMA,
        ],
    )


def _prep_edges(ei, N_dst, nw):
    e = ei.shape[1]
    ep = _ru(e, nw * 128)
    s = jnp.pad(ei[0].astype(jnp.int32), (0, ep - e))
    d = jnp.pad(ei[1].astype(jnp.int32), (0, ep - e), constant_values=N_dst)
    return s, d, ep


def _idx16(s, nw):
    et = s.shape[0] // nw
    return (s.reshape(1, nw, et) * 16
            + jnp.arange(16, dtype=jnp.int32).reshape(16, 1, 1))


NB = 256


def _embed_tc(m, t):
    np_ = m.shape[0]

    def bd(m_ref, t_ref, o_ref):
        o_ref[...] = jnp.dot(m_ref[...], t_ref[...],
                             preferred_element_type=F32)

    return pl.pallas_call(
        bd, grid=(np_ // NB,),
        in_specs=[pl.BlockSpec((NB, 128), lambda i: (i, 0)),
                  pl.BlockSpec((128, H), lambda i: (0, 0))],
        out_specs=pl.BlockSpec((NB, H), lambda i: (i, 0)),
        out_shape=jax.ShapeDtypeStruct((np_, H), F32),
    )(m, t)


def _transform(x, w):
    np_ = x.shape[0]

    def bd(x_ref, w_ref, o_ref):
        o_ref[...] = jnp.dot(x_ref[...], w_ref[...],
                             preferred_element_type=F32)

    return pl.pallas_call(
        bd, grid=(np_ // NB,),
        in_specs=[pl.BlockSpec((NB, H), lambda i: (i, 0)),
                  pl.BlockSpec((H, H), lambda i: (0, 0))],
        out_specs=pl.BlockSpec((NB, H), lambda i: (i, 0)),
        out_shape=jax.ShapeDtypeStruct((np_, H), F32),
    )(x, w)


def _merge_dst(agg, deg, h, wl, wr, bl):
    np_ = h.shape[0]

    def bd(a_ref, d_ref, h_ref, wl_ref, wr_ref, bl_ref, o_ref):
        deg_ = d_ref[0, :, 0:1] + d_ref[1, :, 0:1]
        a = a_ref[...] * (1.0 / jnp.maximum(deg_, 1.0))
        o = (jnp.dot(a, wl_ref[...], preferred_element_type=F32)
             + jnp.dot(h_ref[...], wr_ref[...], preferred_element_type=F32)
             + bl_ref[...])
        o_ref[...] = jnp.maximum(o, 0.0)

    return pl.pallas_call(
        bd, grid=(np_ // NB,),
        in_specs=[pl.BlockSpec((NB, H), lambda i: (i, 0)),
                  pl.BlockSpec((2, NB, 16), lambda i: (0, i, 0)),
                  pl.BlockSpec((NB, H), lambda i: (i, 0)),
                  pl.BlockSpec((H, H), lambda i: (0, 0)),
                  pl.BlockSpec((H, H), lambda i: (0, 0)),
                  pl.BlockSpec((1, H), lambda i: (0, 0))],
        out_specs=pl.BlockSpec((NB, H), lambda i: (i, 0)),
        out_shape=jax.ShapeDtypeStruct((np_, H), F32),
    )(agg, deg, h, wl, wr, bl.reshape(1, H))


def _merge_pin(ac, as_, an, dc, ds_, dn, h, wr_sum, bl_sum):
    np_ = h.shape[0]

    def bd(ac_ref, as_ref, an_ref, dc_ref, ds_ref, dn_ref, h_ref,
           wr_ref, bl_ref, o_ref):
        def term(aref, dref):
            deg_ = dref[0, :, 0:1] + dref[1, :, 0:1]
            return aref[...] * (1.0 / jnp.maximum(deg_, 1.0))
        o = (term(ac_ref, dc_ref) + term(as_ref, ds_ref)
             + term(an_ref, dn_ref)
             + jnp.dot(h_ref[...], wr_ref[...], preferred_element_type=F32)
             + bl_ref[...])
        o_ref[...] = jnp.maximum(o, 0.0)

    sp = pl.BlockSpec((NB, H), lambda i: (i, 0))
    dp = pl.BlockSpec((2, NB, 16), lambda i: (0, i, 0))
    wp = pl.BlockSpec((H, H), lambda i: (0, 0))
    return pl.pallas_call(
        bd, grid=(np_ // NB,),
        in_specs=[sp, sp, sp, dp, dp, dp, sp, wp,
                  pl.BlockSpec((1, H), lambda i: (0, 0))],
        out_specs=sp,
        out_shape=jax.ShapeDtypeStruct((np_, H), F32),
    )(ac, as_, an, dc, ds_, dn, h, wr_sum, bl_sum.reshape(1, H))


def _mlp(h, w1, b1, w2p, b2p):
    np_ = h.shape[0]

    def bd(h_ref, w1_ref, b1_ref, w2_ref, b2_ref, o_ref):
        t = jnp.maximum(
            jnp.dot(h_ref[...], w1_ref[...], preferred_element_type=F32)
            + b1_ref[...], 0.0)
        o_ref[...] = (jnp.dot(t, w2_ref[...], preferred_element_type=F32)
                      + b2_ref[...])

    return pl.pallas_call(
        bd, grid=(np_ // NB,),
        in_specs=[pl.BlockSpec((NB, H), lambda i: (i, 0)),
                  pl.BlockSpec((H, 128), lambda i: (0, 0)),
                  pl.BlockSpec((1, 128), lambda i: (0, 0)),
                  pl.BlockSpec((128, 128), lambda i: (0, 0)),
                  pl.BlockSpec((1, 128), lambda i: (0, 0))],
        out_specs=pl.BlockSpec((NB, 128), lambda i: (i, 0)),
        out_shape=jax.ShapeDtypeStruct((np_, 128), F32),
    )(h, w1, b1.reshape(1, 128), w2p, b2p)


def _multi_hot(x, is_component, np_):
    n = x.shape[0]
    ar = jnp.arange(128, dtype=jnp.int32)[None, :]
    nt = x[:, 0:1].astype(jnp.int32)
    if is_component:
        ct = jnp.zeros_like(nt)
    else:
        ct = jnp.maximum(x[:, 1:2], 0).astype(jnp.int32)
    pt = jnp.maximum(x[:, 2:3], 0).astype(jnp.int32)
    m = ((nt == ar).astype(F32) + (ct + 4 == ar).astype(F32)
         + (pt + 13 == ar).astype(F32))
    return jnp.pad(m, ((0, np_ - n), (0, 0)))


def _flat(h):
    return jnp.reshape(h, (h.shape[0] * 16, 16))


def kernel(x_component, x_pin, x_subcircuit, x_net, edge_cp, edge_pc, edge_sp,
           edge_ps, edge_pn, edge_np, node_type_emb, comp_type_emb,
           pin_type_emb, Wl, bl, Wr, W1, b1, W2, b2):
    t_emb = jnp.concatenate(
        [node_type_emb, comp_type_emb, pin_type_emb,
         jnp.zeros((128 - 26, H), F32)], axis=0)
    m_c = _multi_hot(x_component, True, NP_COMP)
    m_p = _multi_hot(x_pin, False, NP_PIN)
    m_s = _multi_hot(x_subcircuit, False, NP_SUB)
    m_n = _multi_hot(x_net, False, NP_NET)

    s_cp, d_cp, ep_cp = _prep_edges(edge_cp, N_PIN, NT)
    s_sp, d_sp, ep_sp = _prep_edges(edge_sp, N_PIN, NT)
    s_np, d_np, ep_np = _prep_edges(edge_np, N_PIN, NT)
    s_pc, d_pc, ep_pc = _prep_edges(edge_pc, N_COMP, NT)
    s_ps, d_ps, ep_ps = _prep_edges(edge_ps, N_SUB, NT)
    s_pn, d_pn, ep_pn = _prep_edges(edge_pn, N_NET, NT)
    si_cp, si_sp, si_np = _idx16(s_cp, NT), _idx16(s_sp, NT), _idx16(s_np, NT)
    si_pc, si_ps, si_pn = _idx16(s_pc, NT), _idx16(s_ps, NT), _idx16(s_pn, NT)
    di = lambda d: d.reshape(NT, -1, 128)
    d_cp, d_sp, d_np = di(d_cp), di(d_sp), di(d_np)
    d_pc, d_ps, d_pn = di(d_pc), di(d_ps), di(d_pn)

    ones_hbm = jnp.ones((128, 16), F32)
    dg = lambda ei, nd: _prep_edges(ei, nd, 32)
    _, dgd_cp, egp_cp = dg(edge_cp, N_PIN)
    _, dgd_sp, egp_sp = dg(edge_sp, N_PIN)
    _, dgd_np, egp_np = dg(edge_np, N_PIN)
    _, dgd_pc, egp_pc = dg(edge_pc, N_COMP)
    _, dgd_ps, egp_ps = dg(edge_ps, N_SUB)
    _, dgd_pn, egp_pn = dg(edge_pn, N_NET)
    d32 = lambda d: d.reshape(32, -1, 128)
    deg_cp = _make_deg(NP_PIN, egp_cp)(ones_hbm, d32(dgd_cp))
    deg_sp = _make_deg(NP_PIN, egp_sp)(ones_hbm, d32(dgd_sp))
    deg_np = _make_deg(NP_PIN, egp_np)(ones_hbm, d32(dgd_np))
    deg_pc = _make_deg(NP_COMP, egp_pc)(ones_hbm, d32(dgd_pc))
    deg_ps = _make_deg(NP_SUB, egp_ps)(ones_hbm, d32(dgd_ps))
    deg_pn = _make_deg(NP_NET, egp_pn)(ones_hbm, d32(dgd_pn))

    sc_l0 = _make_layer_sc([(NP_PIN, ep_cp), (NP_PIN, ep_sp),
                            (NP_PIN, ep_np), (NP_COMP, ep_pc),
                            (NP_SUB, ep_ps), (NP_NET, ep_pn)])
    sc_l1 = _make_layer_sc([(NP_PIN, ep_cp), (NP_PIN, ep_sp),
                            (NP_PIN, ep_np), (NP_COMP, ep_pc)])
    sc_l2 = _make_layer_sc([(NP_COMP, ep_pc)])

    h_c = _embed_tc(m_c, t_emb)
    h_p = _embed_tc(m_p, t_emb)
    h_s = _embed_tc(m_s, t_emb)
    h_n = _embed_tc(m_n, t_emb)

    for i in range(3):
        wr_pin = Wr[i, 0] + Wr[i, 2] + Wr[i, 5]
        bl_pin = bl[i, 0] + bl[i, 2] + bl[i, 5]
        if i < 2:
            mc = _transform(h_c, Wl[i, 0])
            ms = _transform(h_s, Wl[i, 2])
            mn = _transform(h_n, Wl[i, 5])
        hp_f = _flat(h_p)
        if i == 0:
            agg_cp, agg_sp, agg_np, agg_pc, agg_ps, agg_pn = sc_l0(
                _flat(mc), _flat(ms), _flat(mn), hp_f, hp_f, hp_f,
                si_cp, si_sp, si_np, si_pc, si_ps, si_pn,
                d_cp, d_sp, d_np, d_pc, d_ps, d_pn)
        elif i == 1:
            agg_cp, agg_sp, agg_np, agg_pc = sc_l1(
                _flat(mc), _flat(ms), _flat(mn), hp_f,
                si_cp, si_sp, si_np, si_pc,
                d_cp, d_sp, d_np, d_pc)
        else:
            (agg_pc,) = sc_l2(hp_f, si_pc, d_pc)

        h_c = _merge_dst(agg_pc, deg_pc, h_c, Wl[i, 1], Wr[i, 1], bl[i, 1])
        if i == 0:
            h_s = _merge_dst(agg_ps, deg_ps, h_s, Wl[i, 3], Wr[i, 3],
                             bl[i, 3])
            h_n = _merge_dst(agg_pn, deg_pn, h_n, Wl[i, 4], Wr[i, 4],
                             bl[i, 4])
        if i < 2:
            h_p = _merge_pin(agg_cp, agg_sp, agg_np, deg_cp, deg_sp, deg_np,
                             h_p, wr_pin, bl_pin)

    w2p = jnp.pad(W2, ((0, 0), (0, 118)))
    b2p = jnp.pad(b2, (0, 118)).reshape(1, 128)
    out = _mlp(h_c, W1, b1, w2p, b2p)
    return out[:N_COMP, :10]


# speedup vs baseline: 1.0841x; 1.0841x over previous
import jax
import jax.numpy as jnp
from jax import lax
from jax.experimental import pallas as pl
from jax.experimental.pallas import tpu as pltpu
from jax.experimental.pallas import tpu_sc as plsc

H = 256
F32 = jnp.float32
NT = 16

N_COMP, N_PIN, N_SUB, N_NET = 10000, 50000, 2000, 20000
NP_COMP, NP_PIN, NP_SUB, NP_NET = 10240, 51200, 2048, 20480


def _ru(x, m):
    return (x + m - 1) // m * m


def _make_layer_sc(rels):
    mesh = plsc.VectorSubcoreMesh(core_axis_name="c", subcore_axis_name="s")
    max_et = max(ep // NT for _, ep in rels)
    max_nbt = max_et // 128
    ZR = 64

    def body(*refs):
        n = len(rels)
        srcs = refs[0:n]
        sidxs = refs[n:2 * n]
        didxs = refs[2 * n:3 * n]
        outs = refs[3 * n:4 * n]
        (sidx_v, didx_v, r0_v, r1_v, r2_v, zbuf_v, acc_sh,
         gsem, ssem, zsem) = refs[4 * n:]
        rows = (r0_v, r1_v, r2_v)
        cid = lax.axis_index("c")
        sid = lax.axis_index("s")
        zv = jnp.zeros((16,), F32)

        def zb_fill(i, carry):
            zbuf_v[i // 2, pl.ds((i % 2) * 16, 16)] = zv
            return carry
        lax.fori_loop(0, ZR * 2, zb_fill, 0)

        for r in range(n):
            N_acc, E_pad = rels[r]
            et = E_pad // NT
            nblk = et // 128
            stripe = N_acc // NT
            zrows = min(stripe, ZR)
            nzc = stripe // zrows
            base = cid * 4

            pltpu.sync_copy(didxs[r].at[sid], didx_v.at[pl.ds(0, nblk)])

            def gather(k):
                return pltpu.async_copy(
                    srcs[r].at[sidx_v.at[pl.ds(k * 128, 128)]],
                    rows[k % 3], gsem)

            def scatter(k):
                return pltpu.async_copy(
                    rows[k % 3], acc_sh.at[didx_v.at[k]], ssem, add=True)

            def chunk_body(j, carry):
                cc = base + j
                pltpu.sync_copy(sidxs[r].at[cc, sid],
                                sidx_v.at[pl.ds(0, et)])
                gd = [gather(k) for k in range(min(2, nblk))]
                gd += [None] * (nblk - len(gd))
                cp = base + lax.rem(j + 3, 4)
                pltpu.sync_copy(
                    acc_sh.at[pl.ds(sid * stripe, stripe)],
                    outs[r].at[pl.ds(sid * stripe, stripe),
                               pl.ds(cp * 32, 32)])
                zd = [pltpu.async_copy(
                    zbuf_v.at[pl.ds(0, zrows)],
                    acc_sh.at[pl.ds(sid * stripe + z * zrows, zrows)], zsem)
                    for z in range(nzc)]
                for d in zd:
                    d.wait()
                plsc.subcore_barrier()
                sd = [None] * nblk
                for k in range(nblk):
                    gd[k].wait()
                    sd[k] = scatter(k)
                    nx = k + 2
                    if nx < nblk:
                        if nx - 3 >= 0:
                            sd[nx - 3].wait()
                        gd[nx] = gather(nx)
                for k in range(max(0, nblk - 3), nblk):
                    sd[k].wait()
                plsc.subcore_barrier()
                return carry

            lax.fori_loop(0, 4, chunk_body, 0)
            pltpu.sync_copy(
                acc_sh.at[pl.ds(sid * stripe, stripe)],
                outs[r].at[pl.ds(sid * stripe, stripe),
                           pl.ds((base + 3) * 32, 32)])
            plsc.subcore_barrier()

    return pl.kernel(
        body,
        out_type=tuple(jax.ShapeDtypeStruct((na, H), F32) for na, _ in rels),
        mesh=mesh,
        compiler_params=pltpu.CompilerParams(use_tc_tiling_on_sc=False),
        scratch_types=[
            pltpu.VMEM((max_et,), jnp.int32),
            pltpu.VMEM((max_nbt, 128), jnp.int32),
            pltpu.VMEM((128, 32), F32),
            pltpu.VMEM((128, 32), F32),
            pltpu.VMEM((128, 32), F32),
            pltpu.VMEM((64, 32), F32),
            pltpu.VMEM_SHARED((max(na for na, _ in rels), 32), F32),
            pltpu.SemaphoreType.DMA,
            pltpu.SemaphoreType.DMA,
            pltpu.SemaphoreType.DMA,
        ],
    )


def _make_deg(N_acc, E_pad):
    nw = 32
    NBt = E_pad // nw // 128
    stripe = N_acc // NT
    zrows = min(stripe, 640)
    nzc = stripe // zrows
    mesh = plsc.VectorSubcoreMesh(core_axis_name="c", subcore_axis_name="s")

    def body(ones_hbm, didx_hbm, out_hbm, didx_v, rows_v, zbuf_v, acc_sh,
             gsem, ssem):
        cid = lax.axis_index("c")
        sid = lax.axis_index("s")
        w = sid * 2 + cid
        zv = jnp.zeros((16,), F32)

        def zb_fill(i, carry):
            zbuf_v[i, :] = zv
            return carry
        lax.fori_loop(0, zrows, zb_fill, 0)

        pltpu.sync_copy(didx_hbm.at[w], didx_v)
        for z in range(nzc):
            pltpu.sync_copy(
                zbuf_v.at[pl.ds(0, zrows)],
                acc_sh.at[pl.ds(sid * stripe + z * zrows, zrows)])
        plsc.subcore_barrier()
        pltpu.sync_copy(ones_hbm.at[pl.ds(0, 128)], rows_v)
        sd = []
        for b in range(NBt):
            sd.append(pltpu.async_copy(
                rows_v, acc_sh.at[didx_v.at[b]], ssem, add=True))
        for d in sd:
            d.wait()
        plsc.subcore_barrier()
        pltpu.sync_copy(
            acc_sh.at[pl.ds(sid * stripe, stripe)],
            out_hbm.at[cid, pl.ds(sid * stripe, stripe)])

    return pl.kernel(
        body,
        out_type=jax.ShapeDtypeStruct((2, N_acc, 16), F32),
        mesh=mesh,
        compiler_params=pltpu.CompilerParams(use_tc_tiling_on_sc=False),
        scratch_types=[
            pltpu.VMEM((NBt, 128), jnp.int32),
            pltpu.VMEM((128, 16), F32),
            pltpu.VMEM((640, 16), F32),
            pltpu.VMEM_SHARED((N_acc, 16), F32),
            pltpu.SemaphoreType.DMA,
            pltpu.SemaphoreType.DMA,
        ],
    )


def _prep_edges(ei, N_dst, nw):
    e = ei.shape[1]
    ep = _ru(e, nw * 128)
    s = jnp.pad(ei[0].astype(jnp.int32), (0, ep - e))
    d = jnp.pad(ei[1].astype(jnp.int32), (0, ep - e), constant_values=N_dst)
    return s, d, ep


def _idx16(s, nw):
    et = s.shape[0] // nw
    return (s.reshape(1, nw, et) * 8
            + jnp.arange(8, dtype=jnp.int32).reshape(8, 1, 1))


NB = 256


def _embed_tc(m, t):
    np_ = m.shape[0]

    def bd(m_ref, t_ref, o_ref):
        o_ref[...] = jnp.dot(m_ref[...], t_ref[...],
                             preferred_element_type=F32)

    return pl.pallas_call(
        bd, grid=(np_ // NB,),
        in_specs=[pl.BlockSpec((NB, 128), lambda i: (i, 0)),
                  pl.BlockSpec((128, H), lambda i: (0, 0))],
        out_specs=pl.BlockSpec((NB, H), lambda i: (i, 0)),
        out_shape=jax.ShapeDtypeStruct((np_, H), F32),
    )(m, t)


def _transform(x, w):
    np_ = x.shape[0]

    def bd(x_ref, w_ref, o_ref):
        o_ref[...] = jnp.dot(x_ref[...], w_ref[...],
                             preferred_element_type=F32)

    return pl.pallas_call(
        bd, grid=(np_ // NB,),
        in_specs=[pl.BlockSpec((NB, H), lambda i: (i, 0)),
                  pl.BlockSpec((H, H), lambda i: (0, 0))],
        out_specs=pl.BlockSpec((NB, H), lambda i: (i, 0)),
        out_shape=jax.ShapeDtypeStruct((np_, H), F32),
    )(x, w)


def _merge_dst(agg, deg, h, wl, wr, bl):
    np_ = h.shape[0]

    def bd(a_ref, d_ref, h_ref, wl_ref, wr_ref, bl_ref, o_ref):
        deg_ = d_ref[0, :, 0:1] + d_ref[1, :, 0:1]
        a = a_ref[...] * (1.0 / jnp.maximum(deg_, 1.0))
        o = (jnp.dot(a, wl_ref[...], preferred_element_type=F32)
             + jnp.dot(h_ref[...], wr_ref[...], preferred_element_type=F32)
             + bl_ref[...])
        o_ref[...] = jnp.maximum(o, 0.0)

    return pl.pallas_call(
        bd, grid=(np_ // NB,),
        in_specs=[pl.BlockSpec((NB, H), lambda i: (i, 0)),
                  pl.BlockSpec((2, NB, 16), lambda i: (0, i, 0)),
                  pl.BlockSpec((NB, H), lambda i: (i, 0)),
                  pl.BlockSpec((H, H), lambda i: (0, 0)),
                  pl.BlockSpec((H, H), lambda i: (0, 0)),
                  pl.BlockSpec((1, H), lambda i: (0, 0))],
        out_specs=pl.BlockSpec((NB, H), lambda i: (i, 0)),
        out_shape=jax.ShapeDtypeStruct((np_, H), F32),
    )(agg, deg, h, wl, wr, bl.reshape(1, H))


def _merge_pin(ac, as_, an, dc, ds_, dn, h, wr_sum, bl_sum):
    np_ = h.shape[0]

    def bd(ac_ref, as_ref, an_ref, dc_ref, ds_ref, dn_ref, h_ref,
           wr_ref, bl_ref, o_ref):
        def term(aref, dref):
            deg_ = dref[0, :, 0:1] + dref[1, :, 0:1]
            return aref[...] * (1.0 / jnp.maximum(deg_, 1.0))
        o = (term(ac_ref, dc_ref) + term(as_ref, ds_ref)
             + term(an_ref, dn_ref)
             + jnp.dot(h_ref[...], wr_ref[...], preferred_element_type=F32)
             + bl_ref[...])
        o_ref[...] = jnp.maximum(o, 0.0)

    sp = pl.BlockSpec((NB, H), lambda i: (i, 0))
    dp = pl.BlockSpec((2, NB, 16), lambda i: (0, i, 0))
    wp = pl.BlockSpec((H, H), lambda i: (0, 0))
    return pl.pallas_call(
        bd, grid=(np_ // NB,),
        in_specs=[sp, sp, sp, dp, dp, dp, sp, wp,
                  pl.BlockSpec((1, H), lambda i: (0, 0))],
        out_specs=sp,
        out_shape=jax.ShapeDtypeStruct((np_, H), F32),
    )(ac, as_, an, dc, ds_, dn, h, wr_sum, bl_sum.reshape(1, H))


def _mlp(h, w1, b1, w2p, b2p):
    np_ = h.shape[0]

    def bd(h_ref, w1_ref, b1_ref, w2_ref, b2_ref, o_ref):
        t = jnp.maximum(
            jnp.dot(h_ref[...], w1_ref[...], preferred_element_type=F32)
            + b1_ref[...], 0.0)
        o_ref[...] = (jnp.dot(t, w2_ref[...], preferred_element_type=F32)
                      + b2_ref[...])

    return pl.pallas_call(
        bd, grid=(np_ // NB,),
        in_specs=[pl.BlockSpec((NB, H), lambda i: (i, 0)),
                  pl.BlockSpec((H, 128), lambda i: (0, 0)),
                  pl.BlockSpec((1, 128), lambda i: (0, 0)),
                  pl.BlockSpec((128, 128), lambda i: (0, 0)),
                  pl.BlockSpec((1, 128), lambda i: (0, 0))],
        out_specs=pl.BlockSpec((NB, 128), lambda i: (i, 0)),
        out_shape=jax.ShapeDtypeStruct((np_, 128), F32),
    )(h, w1, b1.reshape(1, 128), w2p, b2p)


def _multi_hot(x, is_component, np_):
    n = x.shape[0]
    ar = jnp.arange(128, dtype=jnp.int32)[None, :]
    nt = x[:, 0:1].astype(jnp.int32)
    if is_component:
        ct = jnp.zeros_like(nt)
    else:
        ct = jnp.maximum(x[:, 1:2], 0).astype(jnp.int32)
    pt = jnp.maximum(x[:, 2:3], 0).astype(jnp.int32)
    m = ((nt == ar).astype(F32) + (ct + 4 == ar).astype(F32)
         + (pt + 13 == ar).astype(F32))
    return jnp.pad(m, ((0, np_ - n), (0, 0)))


def _flat(h):
    return jnp.reshape(h, (h.shape[0] * 8, 32))


def kernel(x_component, x_pin, x_subcircuit, x_net, edge_cp, edge_pc, edge_sp,
           edge_ps, edge_pn, edge_np, node_type_emb, comp_type_emb,
           pin_type_emb, Wl, bl, Wr, W1, b1, W2, b2):
    t_emb = jnp.concatenate(
        [node_type_emb, comp_type_emb, pin_type_emb,
         jnp.zeros((128 - 26, H), F32)], axis=0)
    m_c = _multi_hot(x_component, True, NP_COMP)
    m_p = _multi_hot(x_pin, False, NP_PIN)
    m_s = _multi_hot(x_subcircuit, False, NP_SUB)
    m_n = _multi_hot(x_net, False, NP_NET)

    s_cp, d_cp, ep_cp = _prep_edges(edge_cp, N_PIN, NT)
    s_sp, d_sp, ep_sp = _prep_edges(edge_sp, N_PIN, NT)
    s_np, d_np, ep_np = _prep_edges(edge_np, N_PIN, NT)
    s_pc, d_pc, ep_pc = _prep_edges(edge_pc, N_COMP, NT)
    s_ps, d_ps, ep_ps = _prep_edges(edge_ps, N_SUB, NT)
    s_pn, d_pn, ep_pn = _prep_edges(edge_pn, N_NET, NT)
    si_cp, si_sp, si_np = _idx16(s_cp, NT), _idx16(s_sp, NT), _idx16(s_np, NT)
    si_pc, si_ps, si_pn = _idx16(s_pc, NT), _idx16(s_ps, NT), _idx16(s_pn, NT)
    di = lambda d: d.reshape(NT, -1, 128)
    d_cp, d_sp, d_np = di(d_cp), di(d_sp), di(d_np)
    d_pc, d_ps, d_pn = di(d_pc), di(d_ps), di(d_pn)

    ones_hbm = jnp.ones((128, 16), F32)
    dg = lambda ei, nd: _prep_edges(ei, nd, 32)
    _, dgd_cp, egp_cp = dg(edge_cp, N_PIN)
    _, dgd_sp, egp_sp = dg(edge_sp, N_PIN)
    _, dgd_np, egp_np = dg(edge_np, N_PIN)
    _, dgd_pc, egp_pc = dg(edge_pc, N_COMP)
    _, dgd_ps, egp_ps = dg(edge_ps, N_SUB)
    _, dgd_pn, egp_pn = dg(edge_pn, N_NET)
    d32 = lambda d: d.reshape(32, -1, 128)
    deg_cp = _make_deg(NP_PIN, egp_cp)(ones_hbm, d32(dgd_cp))
    deg_sp = _make_deg(NP_PIN, egp_sp)(ones_hbm, d32(dgd_sp))
    deg_np = _make_deg(NP_PIN, egp_np)(ones_hbm, d32(dgd_np))
    deg_pc = _make_deg(NP_COMP, egp_pc)(ones_hbm, d32(dgd_pc))
    deg_ps = _make_deg(NP_SUB, egp_ps)(ones_hbm, d32(dgd_ps))
    deg_pn = _make_deg(NP_NET, egp_pn)(ones_hbm, d32(dgd_pn))

    sc_l0 = _make_layer_sc([(NP_PIN, ep_cp), (NP_PIN, ep_sp),
                            (NP_PIN, ep_np), (NP_COMP, ep_pc),
                            (NP_SUB, ep_ps), (NP_NET, ep_pn)])
    sc_l1 = _make_layer_sc([(NP_PIN, ep_cp), (NP_PIN, ep_sp),
                            (NP_PIN, ep_np), (NP_COMP, ep_pc)])
    sc_l2 = _make_layer_sc([(NP_COMP, ep_pc)])

    h_c = _embed_tc(m_c, t_emb)
    h_p = _embed_tc(m_p, t_emb)
    h_s = _embed_tc(m_s, t_emb)
    h_n = _embed_tc(m_n, t_emb)

    for i in range(3):
        wr_pin = Wr[i, 0] + Wr[i, 2] + Wr[i, 5]
        bl_pin = bl[i, 0] + bl[i, 2] + bl[i, 5]
        if i < 2:
            mc = _transform(h_c, Wl[i, 0])
            ms = _transform(h_s, Wl[i, 2])
            mn = _transform(h_n, Wl[i, 5])
        hp_f = _flat(h_p)
        if i == 0:
            agg_cp, agg_sp, agg_np, agg_pc, agg_ps, agg_pn = sc_l0(
                _flat(mc), _flat(ms), _flat(mn), hp_f, hp_f, hp_f,
                si_cp, si_sp, si_np, si_pc, si_ps, si_pn,
                d_cp, d_sp, d_np, d_pc, d_ps, d_pn)
        elif i == 1:
            agg_cp, agg_sp, agg_np, agg_pc = sc_l1(
                _flat(mc), _flat(ms), _flat(mn), hp_f,
                si_cp, si_sp, si_np, si_pc,
                d_cp, d_sp, d_np, d_pc)
        else:
            (agg_pc,) = sc_l2(hp_f, si_pc, d_pc)

        h_c = _merge_dst(agg_pc, deg_pc, h_c, Wl[i, 1], Wr[i, 1], bl[i, 1])
        if i == 0:
            h_s = _merge_dst(agg_ps, deg_ps, h_s, Wl[i, 3], Wr[i, 3],
                             bl[i, 3])
            h_n = _merge_dst(agg_pn, deg_pn, h_n, Wl[i, 4], Wr[i, 4],
                             bl[i, 4])
        if i < 2:
            h_p = _merge_pin(agg_cp, agg_sp, agg_np, deg_cp, deg_sp, deg_np,
                             h_p, wr_pin, bl_pin)

    w2p = jnp.pad(W2, ((0, 0), (0, 118)))
    b2p = jnp.pad(b2, (0, 118)).reshape(1, 128)
    out = _mlp(h_c, W1, b1, w2p, b2p)
    return out[:N_COMP, :10]
